# Initial kernel scaffold; baseline (speedup 1.0000x reference)
import jax
import jax.numpy as jnp
from jax.experimental import pallas as pl


def kernel(x, edge_index, W0, b0, W1, b1, W2, b2, g0, bt0, g1, bt1, g2, bt2):
    raise NotImplementedError("write your pallas kernel here")


# speedup vs baseline: 9.3043x; 9.3043x over previous
import functools

import jax
import jax.numpy as jnp
from jax import lax
from jax.experimental import pallas as pl
from jax.experimental.pallas import tpu as pltpu
from jax.experimental.pallas import tpu_sc as plsc

N = 10000
E = 320000
D = 128
NW = 32
EPW = E // NW
CH = 80
NCHUNK = EPW // CH
NROWCHUNK = N // CH
BR = 1000


def _make_sc_agg(with_cnt):
    mesh = plsc.VectorSubcoreMesh(
        core_axis_name="c", subcore_axis_name="s", num_cores=2, num_subcores=16)
    out_type = jax.ShapeDtypeStruct((2, N, D), jnp.float32)
    if with_cnt:
        out_type = [out_type, jax.ShapeDtypeStruct((2, N, D), jnp.float32)]

    @functools.partial(
        pl.kernel,
        out_type=out_type,
        mesh=mesh,
        scratch_types=[
            pltpu.VMEM((NCHUNK, CH), jnp.int32),
            pltpu.VMEM((NCHUNK, CH), jnp.int32),
            pltpu.VMEM((CH, D), jnp.float32),
            pltpu.VMEM_SHARED((N, D), jnp.float32),
        ],
    )
    def agg(g_hbm, src_hbm, dst_hbm, z_hbm, ones_hbm, *rest):
        if with_cnt:
            out_hbm, cnt_hbm, src_v, dst_v, rows_v, acc_sh = rest
        else:
            out_hbm, src_v, dst_v, rows_v, acc_sh = rest
        c = lax.axis_index("c")
        s = lax.axis_index("s")
        wid = c * 16 + s
        pltpu.sync_copy(src_hbm.at[wid], src_v)
        pltpu.sync_copy(dst_hbm.at[wid], dst_v)
        pltpu.sync_copy(z_hbm, rows_v)
        nz = jnp.where(s < NROWCHUNK % 16, NROWCHUNK // 16 + 1, NROWCHUNK // 16)

        def zacc(t, carry):
            pltpu.sync_copy(rows_v, acc_sh.at[pl.ds((s + 16 * t) * CH, CH)])
            return carry
        lax.fori_loop(0, nz, zacc, 0)
        plsc.subcore_barrier()

        def chunk(j, carry):
            pltpu.sync_copy(g_hbm.at[src_v.at[j]], rows_v)
            pltpu.sync_copy(rows_v, acc_sh.at[dst_v.at[j]], add=True)
            return carry
        lax.fori_loop(0, NCHUNK, chunk, 0)
        plsc.subcore_barrier()

        def wout(t, carry):
            off = (s + 16 * t) * CH
            pltpu.sync_copy(acc_sh.at[pl.ds(off, CH)],
                            out_hbm.at[c, pl.ds(off, CH)])
            return carry
        lax.fori_loop(0, nz, wout, 0)

        if with_cnt:
            pltpu.sync_copy(z_hbm, rows_v)
            lax.fori_loop(0, nz, zacc, 0)
            pltpu.sync_copy(ones_hbm, rows_v)
            plsc.subcore_barrier()

            def cchunk(j, carry):
                pltpu.sync_copy(rows_v, acc_sh.at[dst_v.at[j]], add=True)
                return carry
            lax.fori_loop(0, NCHUNK, cchunk, 0)
            plsc.subcore_barrier()

            def wcnt(t, carry):
                off = (s + 16 * t) * CH
                pltpu.sync_copy(acc_sh.at[pl.ds(off, CH)],
                                cnt_hbm.at[c, pl.ds(off, CH)])
                return carry
            lax.fori_loop(0, nz, wcnt, 0)

    return agg


_AGG_CNT = _make_sc_agg(True)
_AGG = _make_sc_agg(False)


def _linear0(x, wt, b):
    def body(x_ref, wt_ref, b_ref, o_ref):
        o_ref[...] = jnp.dot(x_ref[...], wt_ref[...],
                             preferred_element_type=jnp.float32) + b_ref[...]

    return pl.pallas_call(
        body,
        grid=(N // BR,),
        in_specs=[
            pl.BlockSpec((BR, D), lambda i: (i, 0)),
            pl.BlockSpec((D, D), lambda i: (0, 0)),
            pl.BlockSpec((1, D), lambda i: (0, 0)),
        ],
        out_specs=pl.BlockSpec((BR, D), lambda i: (i, 0)),
        out_shape=jax.ShapeDtypeStruct((N, D), jnp.float32),
    )(x, wt, b)


def _fused(p, cnt, g, gam, bet, wt=None, b=None):
    matmul = wt is not None

    def body(p_ref, c_ref, g_ref, gam_ref, bet_ref, *rest):
        pv = p_ref[...]
        ssum = pv[0] + pv[1] + g_ref[...]
        cv = c_ref[...]
        den = cv[0, :, 0:1] + cv[1, :, 0:1] + 1.0
        a = jnp.maximum(ssum / den, 0.0)
        mu = jnp.mean(a, axis=1, keepdims=True)
        var = jnp.mean(jnp.square(a - mu), axis=1, keepdims=True)
        hn = (a - mu) / jnp.sqrt(var + 1e-5) * gam_ref[...] + bet_ref[...]
        if matmul:
            wt_ref, b_ref, o_ref = rest
            o_ref[...] = jnp.dot(hn, wt_ref[...],
                                 preferred_element_type=jnp.float32) + b_ref[...]
        else:
            (o_ref,) = rest
            o_ref[...] = hn

    in_specs = [
        pl.BlockSpec((2, BR, D), lambda i: (0, i, 0)),
        pl.BlockSpec((2, BR, D), lambda i: (0, i, 0)),
        pl.BlockSpec((BR, D), lambda i: (i, 0)),
        pl.BlockSpec((1, D), lambda i: (0, 0)),
        pl.BlockSpec((1, D), lambda i: (0, 0)),
    ]
    args = [p, cnt, g, gam, bet]
    if matmul:
        in_specs += [pl.BlockSpec((D, D), lambda i: (0, 0)),
                     pl.BlockSpec((1, D), lambda i: (0, 0))]
        args += [wt, b]
    return pl.pallas_call(
        body,
        grid=(N // BR,),
        in_specs=in_specs,
        out_specs=pl.BlockSpec((BR, D), lambda i: (i, 0)),
        out_shape=jax.ShapeDtypeStruct((N, D), jnp.float32),
    )(*args)


def kernel(x, edge_index, W0, b0, W1, b1, W2, b2, g0, bt0, g1, bt1, g2, bt2):
    src3 = edge_index[0].reshape(NW, NCHUNK, CH)
    dst3 = edge_index[1].reshape(NW, NCHUNK, CH)
    z = jnp.zeros((CH, D), jnp.float32)
    ones = jnp.ones((CH, D), jnp.float32)
    b0r, b1r, b2r = b0.reshape(1, D), b1.reshape(1, D), b2.reshape(1, D)
    g0r, g1r, g2r = g0.reshape(1, D), g1.reshape(1, D), g2.reshape(1, D)
    bt0r, bt1r, bt2r = bt0.reshape(1, D), bt1.reshape(1, D), bt2.reshape(1, D)

    ga = _linear0(x, W0.T, b0r)
    p0, cnt = _AGG_CNT(ga, src3, dst3, z, ones)
    gb = _fused(p0, cnt, ga, g0r, bt0r, W1.T, b1r)
    p1 = _AGG(gb, src3, dst3, z, ones)
    gc = _fused(p1, cnt, gb, g1r, bt1r, W2.T, b2r)
    p2 = _AGG(gc, src3, dst3, z, ones)
    return _fused(p2, cnt, gc, g2r, bt2r)


# speedup vs baseline: 12.0280x; 1.2927x over previous
import functools

import jax
import jax.numpy as jnp
from jax import lax
from jax.experimental import pallas as pl
from jax.experimental.pallas import tpu as pltpu
from jax.experimental.pallas import tpu_sc as plsc

N = 10000
E = 320000
D = 128
NW = 32
EPW = E // NW
CH = 80
NCHUNK = EPW // CH
NROWCHUNK = N // CH
BR = 1000


def _make_sc_agg(with_cnt):
    mesh = plsc.VectorSubcoreMesh(
        core_axis_name="c", subcore_axis_name="s", num_cores=2, num_subcores=16)
    out_type = jax.ShapeDtypeStruct((2, N, D), jnp.float32)
    if with_cnt:
        out_type = [out_type, jax.ShapeDtypeStruct((2, N, D), jnp.float32)]

    @functools.partial(
        pl.kernel,
        out_type=out_type,
        mesh=mesh,
        scratch_types=[
            pltpu.VMEM((2, 1, CH), jnp.int32),
            pltpu.VMEM((NCHUNK, CH), jnp.int32),
            pltpu.VMEM((2, CH, D), jnp.float32),
            pltpu.VMEM_SHARED((N, D), jnp.float32),
            pltpu.SemaphoreType.DMA,
            pltpu.SemaphoreType.DMA,
            pltpu.SemaphoreType.DMA,
            pltpu.SemaphoreType.DMA,
        ],
    )
    def agg(g_hbm, src_hbm, dst_hbm, z_hbm, ones_hbm, *rest):
        if with_cnt:
            (out_hbm, cnt_hbm, src_v, dst_v, rows_v, acc_sh,
             semg0, semg1, semi0, semi1) = rest
        else:
            (out_hbm, src_v, dst_v, rows_v, acc_sh,
             semg0, semg1, semi0, semi1) = rest
        semg = (semg0, semg1)
        semi = (semi0, semi1)
        c = lax.axis_index("c")
        s = lax.axis_index("s")
        wid = c * 16 + s
        pltpu.sync_copy(dst_hbm.at[wid], dst_v)
        pltpu.sync_copy(z_hbm, rows_v.at[0])
        nz = jnp.where(s < NROWCHUNK % 16, NROWCHUNK // 16 + 1, NROWCHUNK // 16)

        def zacc(t, carry):
            pltpu.sync_copy(rows_v.at[0], acc_sh.at[pl.ds((s + 16 * t) * CH, CH)])
            return carry
        lax.fori_loop(0, nz, zacc, 0)
        plsc.subcore_barrier()

        pltpu.sync_copy(src_hbm.at[wid, 0], src_v.at[0])
        pltpu.async_copy(g_hbm.at[src_v.at[0, 0]], rows_v.at[0], semg0)
        pltpu.async_copy(src_hbm.at[wid, 1], src_v.at[1], semi1)

        def chunk(j, carry):
            for b in (0, 1):
                @pl.when(lax.rem(j, 2) == b)
                def _():
                    o = 1 - b

                    @pl.when(j + 1 < NCHUNK)
                    def _():
                        pltpu.make_async_copy(src_hbm.at[wid, j + 1],
                                              src_v.at[o], semi[o]).wait()
                        pltpu.async_copy(g_hbm.at[src_v.at[o, 0]],
                                         rows_v.at[o], semg[o])
                    pltpu.make_async_copy(g_hbm.at[src_v.at[b, 0]],
                                          rows_v.at[b], semg[b]).wait()
                    pltpu.sync_copy(rows_v.at[b], acc_sh.at[dst_v.at[j]],
                                    add=True)

                    @pl.when(j + 2 < NCHUNK)
                    def _():
                        pltpu.async_copy(src_hbm.at[wid, j + 2],
                                         src_v.at[b], semi[b])
            return carry
        lax.fori_loop(0, NCHUNK, chunk, 0)
        plsc.subcore_barrier()

        def wout(t, carry):
            off = (s + 16 * t) * CH
            pltpu.sync_copy(acc_sh.at[pl.ds(off, CH)],
                            out_hbm.at[c, pl.ds(off, CH)])
            return carry
        lax.fori_loop(0, nz, wout, 0)

        if with_cnt:
            pltpu.sync_copy(z_hbm, rows_v.at[0])
            lax.fori_loop(0, nz, zacc, 0)
            pltpu.sync_copy(ones_hbm, rows_v.at[0])
            plsc.subcore_barrier()

            def cchunk(j, carry):
                pltpu.sync_copy(rows_v.at[0], acc_sh.at[dst_v.at[j]], add=True)
                return carry
            lax.fori_loop(0, NCHUNK, cchunk, 0)
            plsc.subcore_barrier()

            def wcnt(t, carry):
                off = (s + 16 * t) * CH
                pltpu.sync_copy(acc_sh.at[pl.ds(off, CH)],
                                cnt_hbm.at[c, pl.ds(off, CH)])
                return carry
            lax.fori_loop(0, nz, wcnt, 0)

    return agg


_AGG_CNT = _make_sc_agg(True)
_AGG = _make_sc_agg(False)


def _linear0(x, wt, b):
    def body(x_ref, wt_ref, b_ref, o_ref):
        o_ref[...] = jnp.dot(x_ref[...], wt_ref[...],
                             preferred_element_type=jnp.float32) + b_ref[...]

    return pl.pallas_call(
        body,
        grid=(N // BR,),
        in_specs=[
            pl.BlockSpec((BR, D), lambda i: (i, 0)),
            pl.BlockSpec((D, D), lambda i: (0, 0)),
            pl.BlockSpec((1, D), lambda i: (0, 0)),
        ],
        out_specs=pl.BlockSpec((BR, D), lambda i: (i, 0)),
        out_shape=jax.ShapeDtypeStruct((N, D), jnp.float32),
    )(x, wt, b)


def _fused(p, cnt, g, gam, bet, wt=None, b=None):
    matmul = wt is not None

    def body(p_ref, c_ref, g_ref, gam_ref, bet_ref, *rest):
        pv = p_ref[...]
        ssum = pv[0] + pv[1] + g_ref[...]
        cv = c_ref[...]
        den = cv[0, :, 0:1] + cv[1, :, 0:1] + 1.0
        a = jnp.maximum(ssum / den, 0.0)
        mu = jnp.mean(a, axis=1, keepdims=True)
        var = jnp.mean(jnp.square(a - mu), axis=1, keepdims=True)
        hn = (a - mu) / jnp.sqrt(var + 1e-5) * gam_ref[...] + bet_ref[...]
        if matmul:
            wt_ref, b_ref, o_ref = rest
            o_ref[...] = jnp.dot(hn, wt_ref[...],
                                 preferred_element_type=jnp.float32) + b_ref[...]
        else:
            (o_ref,) = rest
            o_ref[...] = hn

    in_specs = [
        pl.BlockSpec((2, BR, D), lambda i: (0, i, 0)),
        pl.BlockSpec((2, BR, D), lambda i: (0, i, 0)),
        pl.BlockSpec((BR, D), lambda i: (i, 0)),
        pl.BlockSpec((1, D), lambda i: (0, 0)),
        pl.BlockSpec((1, D), lambda i: (0, 0)),
    ]
    args = [p, cnt, g, gam, bet]
    if matmul:
        in_specs += [pl.BlockSpec((D, D), lambda i: (0, 0)),
                     pl.BlockSpec((1, D), lambda i: (0, 0))]
        args += [wt, b]
    return pl.pallas_call(
        body,
        grid=(N // BR,),
        in_specs=in_specs,
        out_specs=pl.BlockSpec((BR, D), lambda i: (i, 0)),
        out_shape=jax.ShapeDtypeStruct((N, D), jnp.float32),
    )(*args)


def kernel(x, edge_index, W0, b0, W1, b1, W2, b2, g0, bt0, g1, bt1, g2, bt2):
    src3 = edge_index[0].reshape(NW, NCHUNK, 1, CH)
    dst3 = edge_index[1].reshape(NW, NCHUNK, CH)
    z = jnp.zeros((CH, D), jnp.float32)
    ones = jnp.ones((CH, D), jnp.float32)
    b0r, b1r, b2r = b0.reshape(1, D), b1.reshape(1, D), b2.reshape(1, D)
    g0r, g1r, g2r = g0.reshape(1, D), g1.reshape(1, D), g2.reshape(1, D)
    bt0r, bt1r, bt2r = bt0.reshape(1, D), bt1.reshape(1, D), bt2.reshape(1, D)

    ga = _linear0(x, W0.T, b0r)
    p0, cnt = _AGG_CNT(ga, src3, dst3, z, ones)
    gb = _fused(p0, cnt, ga, g0r, bt0r, W1.T, b1r)
    p1 = _AGG(gb, src3, dst3, z, ones)
    gc = _fused(p1, cnt, gb, g1r, bt1r, W2.T, b2r)
    p2 = _AGG(gc, src3, dst3, z, ones)
    return _fused(p2, cnt, gc, g2r, bt2r)


# speedup vs baseline: 13.1783x; 1.0956x over previous
import dataclasses
import functools

import jax
import jax.numpy as jnp
from jax import lax
from jax.experimental import pallas as pl
from jax.experimental.pallas import tpu as pltpu
from jax.experimental.pallas import tpu_sc as plsc

N = 10000
E = 320000
D = 128
NW = 32
EPW = E // NW
CH = 80
NCHUNK = EPW // CH
NROWCHUNK = N // CH
BR = 1000


def _make_sc_agg(with_cnt):
    mesh = plsc.VectorSubcoreMesh(
        core_axis_name="c", subcore_axis_name="s", num_cores=2, num_subcores=16)
    out_type = jax.ShapeDtypeStruct((2, N, D), jnp.float32)
    if with_cnt:
        out_type = [out_type, jax.ShapeDtypeStruct((2, CH, D), jnp.float32)]
    scratch = [
        pltpu.VMEM((2, 1, CH), jnp.int32),
        pltpu.VMEM((NCHUNK, CH), jnp.int32),
        pltpu.VMEM((2, CH, D), jnp.float32),
        pltpu.VMEM_SHARED((N, D), jnp.float32),
        pltpu.SemaphoreType.DMA,
        pltpu.SemaphoreType.DMA,
        pltpu.SemaphoreType.DMA,
        pltpu.SemaphoreType.DMA,
    ]
    if with_cnt:
        scratch.append(pltpu.VMEM((1, CH), jnp.int32))

    @functools.partial(
        pl.kernel,
        out_type=out_type,
        mesh=mesh,
        compiler_params=dataclasses.replace(
            pltpu.CompilerParams(), needs_layout_passes=not with_cnt),
        scratch_types=scratch,
    )
    def agg(g_hbm, src_hbm, dst_hbm, z_hbm, idn_hbm, *rest):
        if with_cnt:
            (out_hbm, cnt_hbm, src_v, dst_v, rows_v, acc_sh,
             semg0, semg1, semi0, semi1, idn_v) = rest
        else:
            (out_hbm, src_v, dst_v, rows_v, acc_sh,
             semg0, semg1, semi0, semi1) = rest
        semg = (semg0, semg1)
        semi = (semi0, semi1)
        c = lax.axis_index("c")
        s = lax.axis_index("s")
        wid = c * 16 + s
        pltpu.sync_copy(dst_hbm.at[wid], dst_v)
        pltpu.sync_copy(z_hbm, rows_v.at[0])
        nz = jnp.where(s < NROWCHUNK % 16, NROWCHUNK // 16 + 1, NROWCHUNK // 16)

        def zacc(t, carry):
            pltpu.sync_copy(rows_v.at[0], acc_sh.at[pl.ds((s + 16 * t) * CH, CH)])
            return carry
        lax.fori_loop(0, nz, zacc, 0)
        plsc.subcore_barrier()

        pltpu.sync_copy(src_hbm.at[wid, 0], src_v.at[0])
        pltpu.async_copy(g_hbm.at[src_v.at[0, 0]], rows_v.at[0], semg0)
        pltpu.async_copy(src_hbm.at[wid, 1], src_v.at[1], semi1)

        def chunk(j, carry):
            for b in (0, 1):
                @pl.when(lax.rem(j, 2) == b)
                def _():
                    o = 1 - b

                    @pl.when(j + 1 < NCHUNK)
                    def _():
                        pltpu.make_async_copy(src_hbm.at[wid, j + 1],
                                              src_v.at[o], semi[o]).wait()
                        pltpu.async_copy(g_hbm.at[src_v.at[o, 0]],
                                         rows_v.at[o], semg[o])
                    pltpu.make_async_copy(g_hbm.at[src_v.at[b, 0]],
                                          rows_v.at[b], semg[b]).wait()
                    pltpu.sync_copy(rows_v.at[b], acc_sh.at[dst_v.at[j]],
                                    add=True)

                    @pl.when(j + 2 < NCHUNK)
                    def _():
                        pltpu.async_copy(src_hbm.at[wid, j + 2],
                                         src_v.at[b], semi[b])
            return carry
        lax.fori_loop(0, NCHUNK, chunk, 0)
        plsc.subcore_barrier()

        def wout(t, carry):
            off = (s + 16 * t) * CH
            pltpu.sync_copy(acc_sh.at[pl.ds(off, CH)],
                            out_hbm.at[c, pl.ds(off, CH)])
            return carry
        lax.fori_loop(0, nz, wout, 0)

        if with_cnt:
            pltpu.sync_copy(z_hbm, rows_v.at[0])
            pltpu.sync_copy(idn_hbm, idn_v)

            @pl.when(s == 0)
            def _():
                pltpu.sync_copy(z_hbm, acc_sh.at[pl.ds(0, CH)])
            hist = rows_v.at[0]
            ones16 = jnp.full((16,), 1.0, jnp.float32)

            def hloop(j, carry):
                for k in range(CH // 16):
                    idx16 = dst_v[j, pl.ds(k * 16, 16)]
                    plsc.addupdate_scatter(
                        hist,
                        [lax.shift_right_logical(idx16, 7),
                         lax.bitwise_and(idx16, 127)],
                        ones16)
                return carry
            lax.fori_loop(0, NCHUNK, hloop, 0)
            plsc.subcore_barrier()
            pltpu.sync_copy(hist, acc_sh.at[idn_v.at[0]], add=True)
            plsc.subcore_barrier()

            @pl.when(s == 0)
            def _():
                pltpu.sync_copy(acc_sh.at[pl.ds(0, CH)], cnt_hbm.at[c])

    return agg


_AGG_CNT = _make_sc_agg(True)
_AGG = _make_sc_agg(False)


def _linear0(x, wt, b):
    def body(x_ref, wt_ref, b_ref, o_ref):
        o_ref[...] = jnp.dot(x_ref[...], wt_ref[...],
                             preferred_element_type=jnp.float32) + b_ref[...]

    return pl.pallas_call(
        body,
        grid=(N // BR,),
        in_specs=[
            pl.BlockSpec((BR, D), lambda i: (i, 0)),
            pl.BlockSpec((D, D), lambda i: (0, 0)),
            pl.BlockSpec((1, D), lambda i: (0, 0)),
        ],
        out_specs=pl.BlockSpec((BR, D), lambda i: (i, 0)),
        out_shape=jax.ShapeDtypeStruct((N, D), jnp.float32),
    )(x, wt, b)


def _fused(p, ca, cb, g, gam, bet, wt=None, b=None):
    matmul = wt is not None

    def body(p_ref, ca_ref, cb_ref, g_ref, gam_ref, bet_ref, *rest):
        pv = p_ref[...]
        ssum = pv[0] + pv[1] + g_ref[...]
        den = ca_ref[...] + cb_ref[...] + 1.0
        a = jnp.maximum(ssum / den, 0.0)
        mu = jnp.mean(a, axis=1, keepdims=True)
        var = jnp.mean(jnp.square(a - mu), axis=1, keepdims=True)
        hn = (a - mu) / jnp.sqrt(var + 1e-5) * gam_ref[...] + bet_ref[...]
        if matmul:
            wt_ref, b_ref, o_ref = rest
            o_ref[...] = jnp.dot(hn, wt_ref[...],
                                 preferred_element_type=jnp.float32) + b_ref[...]
        else:
            (o_ref,) = rest
            o_ref[...] = hn

    in_specs = [
        pl.BlockSpec((2, BR, D), lambda i: (0, i, 0)),
        pl.BlockSpec((BR, 1), lambda i: (i, 0)),
        pl.BlockSpec((BR, 1), lambda i: (i, 0)),
        pl.BlockSpec((BR, D), lambda i: (i, 0)),
        pl.BlockSpec((1, D), lambda i: (0, 0)),
        pl.BlockSpec((1, D), lambda i: (0, 0)),
    ]
    args = [p, ca, cb, g, gam, bet]
    if matmul:
        in_specs += [pl.BlockSpec((D, D), lambda i: (0, 0)),
                     pl.BlockSpec((1, D), lambda i: (0, 0))]
        args += [wt, b]
    return pl.pallas_call(
        body,
        grid=(N // BR,),
        in_specs=in_specs,
        out_specs=pl.BlockSpec((BR, D), lambda i: (i, 0)),
        out_shape=jax.ShapeDtypeStruct((N, D), jnp.float32),
    )(*args)


def kernel(x, edge_index, W0, b0, W1, b1, W2, b2, g0, bt0, g1, bt1, g2, bt2):
    src3 = edge_index[0].reshape(NW, NCHUNK, 1, CH)
    dst3 = edge_index[1].reshape(NW, NCHUNK, CH)
    z = jnp.zeros((CH, D), jnp.float32)
    idn = jnp.arange(CH, dtype=jnp.int32).reshape(1, CH)
    b0r, b1r, b2r = b0.reshape(1, D), b1.reshape(1, D), b2.reshape(1, D)
    g0r, g1r, g2r = g0.reshape(1, D), g1.reshape(1, D), g2.reshape(1, D)
    bt0r, bt1r, bt2r = bt0.reshape(1, D), bt1.reshape(1, D), bt2.reshape(1, D)

    ga = _linear0(x, W0.T, b0r)
    p0, cnt = _AGG_CNT(ga, src3, dst3, z, idn)
    ca = cnt[0].reshape(CH * D, 1)[:N]
    cb = cnt[1].reshape(CH * D, 1)[:N]
    gb = _fused(p0, ca, cb, ga, g0r, bt0r, W1.T, b1r)
    p1 = _AGG(gb, src3, dst3, z, idn)
    gc = _fused(p1, ca, cb, gb, g1r, bt1r, W2.T, b2r)
    p2 = _AGG(gc, src3, dst3, z, idn)
    return _fused(p2, ca, cb, gc, g2r, bt2r)


# speedup vs baseline: 13.2322x; 1.0041x over previous
import dataclasses
import functools

import jax
import jax.numpy as jnp
from jax import lax
from jax.experimental import pallas as pl
from jax.experimental.pallas import tpu as pltpu
from jax.experimental.pallas import tpu_sc as plsc

N = 10000
E = 320000
D = 128
NW = 32
EPW = E // NW
CH = 80
NCHUNK = EPW // CH
NROWCHUNK = N // CH
BR = 1000


def _make_sc_agg(with_cnt):
    mesh = plsc.VectorSubcoreMesh(
        core_axis_name="c", subcore_axis_name="s", num_cores=2, num_subcores=16)
    out_type = jax.ShapeDtypeStruct((2, N, D), jnp.float32)
    if with_cnt:
        out_type = [out_type, jax.ShapeDtypeStruct((2, CH, D), jnp.float32)]
    scratch = [
        pltpu.VMEM((2, 1, CH), jnp.int32),
        pltpu.VMEM((NCHUNK, CH), jnp.int32),
        pltpu.VMEM((2, CH, D), jnp.float32),
        pltpu.VMEM_SHARED((N, D), jnp.float32),
        pltpu.SemaphoreType.DMA,
        pltpu.SemaphoreType.DMA,
        pltpu.SemaphoreType.DMA,
        pltpu.SemaphoreType.DMA,
    ]
    if with_cnt:
        scratch.append(pltpu.VMEM((1, CH), jnp.int32))

    @functools.partial(
        pl.kernel,
        out_type=out_type,
        mesh=mesh,
        compiler_params=dataclasses.replace(
            pltpu.CompilerParams(), needs_layout_passes=not with_cnt),
        scratch_types=scratch,
    )
    def agg(g_hbm, src_hbm, dst_hbm, z_hbm, idn_hbm, *rest):
        if with_cnt:
            (out_hbm, cnt_hbm, src_v, dst_v, rows_v, acc_sh,
             semg0, semg1, semi0, semi1, idn_v) = rest
        else:
            (out_hbm, src_v, dst_v, rows_v, acc_sh,
             semg0, semg1, semi0, semi1) = rest
        semg = (semg0, semg1)
        semi = (semi0, semi1)
        c = lax.axis_index("c")
        s = lax.axis_index("s")
        wid = c * 16 + s
        pltpu.sync_copy(dst_hbm.at[wid], dst_v)
        pltpu.sync_copy(src_hbm.at[wid, 0], src_v.at[0])
        pltpu.async_copy(g_hbm.at[src_v.at[0, 0]], rows_v.at[0], semg0)
        pltpu.async_copy(src_hbm.at[wid, 1], src_v.at[1], semi1)
        pltpu.sync_copy(z_hbm, rows_v.at[1])
        nz = jnp.where(s < NROWCHUNK % 16, NROWCHUNK // 16 + 1, NROWCHUNK // 16)

        def zacc(t, carry):
            pltpu.sync_copy(rows_v.at[1], acc_sh.at[pl.ds((s + 16 * t) * CH, CH)])
            return carry
        lax.fori_loop(0, nz, zacc, 0)
        plsc.subcore_barrier()


        def chunk(j, carry):
            for b in (0, 1):
                @pl.when(lax.rem(j, 2) == b)
                def _():
                    o = 1 - b

                    @pl.when(j + 1 < NCHUNK)
                    def _():
                        pltpu.make_async_copy(src_hbm.at[wid, j + 1],
                                              src_v.at[o], semi[o]).wait()
                        pltpu.async_copy(g_hbm.at[src_v.at[o, 0]],
                                         rows_v.at[o], semg[o])
                    pltpu.make_async_copy(g_hbm.at[src_v.at[b, 0]],
                                          rows_v.at[b], semg[b]).wait()
                    pltpu.sync_copy(rows_v.at[b], acc_sh.at[dst_v.at[j]],
                                    add=True)

                    @pl.when(j + 2 < NCHUNK)
                    def _():
                        pltpu.async_copy(src_hbm.at[wid, j + 2],
                                         src_v.at[b], semi[b])
            return carry
        lax.fori_loop(0, NCHUNK, chunk, 0)
        plsc.subcore_barrier()

        def wout(t, carry):
            off = (s + 16 * t) * CH
            pltpu.sync_copy(acc_sh.at[pl.ds(off, CH)],
                            out_hbm.at[c, pl.ds(off, CH)])
            return carry
        lax.fori_loop(0, nz, wout, 0)

        if with_cnt:
            pltpu.sync_copy(z_hbm, rows_v.at[0])
            pltpu.sync_copy(idn_hbm, idn_v)

            @pl.when(s == 0)
            def _():
                pltpu.sync_copy(z_hbm, acc_sh.at[pl.ds(0, CH)])
            hist = rows_v.at[0]
            ones16 = jnp.full((16,), 1.0, jnp.float32)

            def hloop(j, carry):
                for k in range(CH // 16):
                    idx16 = dst_v[j, pl.ds(k * 16, 16)]
                    plsc.addupdate_scatter(
                        hist,
                        [lax.shift_right_logical(idx16, 7),
                         lax.bitwise_and(idx16, 127)],
                        ones16)
                return carry
            lax.fori_loop(0, NCHUNK, hloop, 0)
            plsc.subcore_barrier()
            pltpu.sync_copy(hist, acc_sh.at[idn_v.at[0]], add=True)
            plsc.subcore_barrier()

            @pl.when(s == 0)
            def _():
                pltpu.sync_copy(acc_sh.at[pl.ds(0, CH)], cnt_hbm.at[c])

    return agg


_AGG_CNT = _make_sc_agg(True)
_AGG = _make_sc_agg(False)


def _linear0(x, wt, b):
    def body(x_ref, wt_ref, b_ref, o_ref):
        o_ref[...] = jnp.dot(x_ref[...], wt_ref[...],
                             preferred_element_type=jnp.float32) + b_ref[...]

    return pl.pallas_call(
        body,
        grid=(N // BR,),
        in_specs=[
            pl.BlockSpec((BR, D), lambda i: (i, 0)),
            pl.BlockSpec((D, D), lambda i: (0, 0)),
            pl.BlockSpec((1, D), lambda i: (0, 0)),
        ],
        out_specs=pl.BlockSpec((BR, D), lambda i: (i, 0)),
        out_shape=jax.ShapeDtypeStruct((N, D), jnp.float32),
    )(x, wt, b)


def _fused(p, ca, cb, g, gam, bet, wt=None, b=None):
    matmul = wt is not None

    def body(p_ref, ca_ref, cb_ref, g_ref, gam_ref, bet_ref, *rest):
        pv = p_ref[...]
        ssum = pv[0] + pv[1] + g_ref[...]
        den = ca_ref[...] + cb_ref[...] + 1.0
        a = jnp.maximum(ssum / den, 0.0)
        mu = jnp.mean(a, axis=1, keepdims=True)
        var = jnp.mean(jnp.square(a - mu), axis=1, keepdims=True)
        hn = (a - mu) / jnp.sqrt(var + 1e-5) * gam_ref[...] + bet_ref[...]
        if matmul:
            wt_ref, b_ref, o_ref = rest
            o_ref[...] = jnp.dot(hn, wt_ref[...],
                                 preferred_element_type=jnp.float32) + b_ref[...]
        else:
            (o_ref,) = rest
            o_ref[...] = hn

    in_specs = [
        pl.BlockSpec((2, BR, D), lambda i: (0, i, 0)),
        pl.BlockSpec((BR, 1), lambda i: (i, 0)),
        pl.BlockSpec((BR, 1), lambda i: (i, 0)),
        pl.BlockSpec((BR, D), lambda i: (i, 0)),
        pl.BlockSpec((1, D), lambda i: (0, 0)),
        pl.BlockSpec((1, D), lambda i: (0, 0)),
    ]
    args = [p, ca, cb, g, gam, bet]
    if matmul:
        in_specs += [pl.BlockSpec((D, D), lambda i: (0, 0)),
                     pl.BlockSpec((1, D), lambda i: (0, 0))]
        args += [wt, b]
    return pl.pallas_call(
        body,
        grid=(N // BR,),
        in_specs=in_specs,
        out_specs=pl.BlockSpec((BR, D), lambda i: (i, 0)),
        out_shape=jax.ShapeDtypeStruct((N, D), jnp.float32),
    )(*args)


def kernel(x, edge_index, W0, b0, W1, b1, W2, b2, g0, bt0, g1, bt1, g2, bt2):
    src3 = edge_index[0].reshape(NW, NCHUNK, 1, CH)
    dst3 = edge_index[1].reshape(NW, NCHUNK, CH)
    z = jnp.zeros((CH, D), jnp.float32)
    idn = jnp.arange(CH, dtype=jnp.int32).reshape(1, CH)
    b0r, b1r, b2r = b0.reshape(1, D), b1.reshape(1, D), b2.reshape(1, D)
    g0r, g1r, g2r = g0.reshape(1, D), g1.reshape(1, D), g2.reshape(1, D)
    bt0r, bt1r, bt2r = bt0.reshape(1, D), bt1.reshape(1, D), bt2.reshape(1, D)

    ga = _linear0(x, W0.T, b0r)
    p0, cnt = _AGG_CNT(ga, src3, dst3, z, idn)
    ca = cnt[0].reshape(CH * D, 1)[:N]
    cb = cnt[1].reshape(CH * D, 1)[:N]
    gb = _fused(p0, ca, cb, ga, g0r, bt0r, W1.T, b1r)
    p1 = _AGG(gb, src3, dst3, z, idn)
    gc = _fused(p1, ca, cb, gb, g1r, bt1r, W2.T, b2r)
    p2 = _AGG(gc, src3, dst3, z, idn)
    return _fused(p2, ca, cb, gc, g2r, bt2r)


# speedup vs baseline: 15.1083x; 1.1418x over previous
import dataclasses
import functools

import jax
import jax.numpy as jnp
from jax import lax
from jax.experimental import pallas as pl
from jax.experimental.pallas import tpu as pltpu
from jax.experimental.pallas import tpu_sc as plsc

N = 10000
E = 320000
D = 128
NW = 32
EPW = E // NW
CH = 80
NCHUNK = EPW // CH
NROWCHUNK = N // CH
BR = 1000


def _make_sc_agg(with_cnt):
    mesh = plsc.VectorSubcoreMesh(
        core_axis_name="c", subcore_axis_name="s", num_cores=2, num_subcores=16)
    out_type = jax.ShapeDtypeStruct((2, N, D), jnp.float32)
    if with_cnt:
        out_type = [out_type, jax.ShapeDtypeStruct((2, CH, D), jnp.float32)]
    scratch = [
        pltpu.VMEM((2, 1, CH), jnp.int32),
        pltpu.VMEM((NCHUNK, CH), jnp.int32),
        pltpu.VMEM((2, CH, D), jnp.float32),
        pltpu.VMEM_SHARED((N, D), jnp.float32),
        pltpu.SemaphoreType.DMA,
        pltpu.SemaphoreType.DMA,
        pltpu.SemaphoreType.DMA,
        pltpu.SemaphoreType.DMA,
        pltpu.SemaphoreType.DMA,
        pltpu.SemaphoreType.DMA,
    ]
    if with_cnt:
        scratch.append(pltpu.VMEM((1, CH), jnp.int32))

    @functools.partial(
        pl.kernel,
        out_type=out_type,
        mesh=mesh,
        compiler_params=dataclasses.replace(
            pltpu.CompilerParams(), needs_layout_passes=not with_cnt),
        scratch_types=scratch,
    )
    def agg(g_hbm, src_hbm, dst_hbm, z_hbm, idn_hbm, *rest):
        if with_cnt:
            (out_hbm, cnt_hbm, src_v, dst_v, rows_v, acc_sh,
             semg0, semg1, semi0, semi1, sems0, sems1, idn_v) = rest
        else:
            (out_hbm, src_v, dst_v, rows_v, acc_sh,
             semg0, semg1, semi0, semi1, sems0, sems1) = rest
        semg = (semg0, semg1)
        semi = (semi0, semi1)
        sems = (sems0, sems1)
        c = lax.axis_index("c")
        s = lax.axis_index("s")
        wid = c * 16 + s
        pltpu.sync_copy(dst_hbm.at[wid], dst_v)
        pltpu.sync_copy(src_hbm.at[wid, 0], src_v.at[0])
        pltpu.async_copy(g_hbm.at[src_v.at[0, 0]], rows_v.at[0], semg0)
        pltpu.async_copy(src_hbm.at[wid, 1], src_v.at[1], semi1)
        pltpu.sync_copy(z_hbm, rows_v.at[1])
        nz = jnp.where(s < NROWCHUNK % 16, NROWCHUNK // 16 + 1, NROWCHUNK // 16)

        def zacc(t, carry):
            pltpu.sync_copy(rows_v.at[1], acc_sh.at[pl.ds((s + 16 * t) * CH, CH)])
            return carry
        lax.fori_loop(0, nz, zacc, 0)
        plsc.subcore_barrier()


        def chunk(j, carry):
            for b in (0, 1):
                @pl.when(lax.rem(j, 2) == b)
                def _():
                    o = 1 - b

                    @pl.when(j + 1 < NCHUNK)
                    def _():
                        pltpu.make_async_copy(src_hbm.at[wid, j + 1],
                                              src_v.at[o], semi[o]).wait()

                        @pl.when(j >= 1)
                        def _():
                            pltpu.make_async_copy(
                                rows_v.at[o], acc_sh.at[dst_v.at[j - 1]],
                                sems[o]).wait()
                        pltpu.async_copy(g_hbm.at[src_v.at[o, 0]],
                                         rows_v.at[o], semg[o])
                    pltpu.make_async_copy(g_hbm.at[src_v.at[b, 0]],
                                          rows_v.at[b], semg[b]).wait()
                    pltpu.async_copy(rows_v.at[b], acc_sh.at[dst_v.at[j]],
                                     sems[b], add=True)

                    @pl.when(j + 2 < NCHUNK)
                    def _():
                        pltpu.async_copy(src_hbm.at[wid, j + 2],
                                         src_v.at[b], semi[b])
            return carry
        lax.fori_loop(0, NCHUNK, chunk, 0)
        pltpu.make_async_copy(rows_v.at[0], acc_sh.at[dst_v.at[NCHUNK - 1]],
                              sems0).wait()
        pltpu.make_async_copy(rows_v.at[1], acc_sh.at[dst_v.at[NCHUNK - 2]],
                              sems1).wait()
        plsc.subcore_barrier()

        def wout(t, carry):
            off = (s + 16 * t) * CH
            pltpu.sync_copy(acc_sh.at[pl.ds(off, CH)],
                            out_hbm.at[c, pl.ds(off, CH)])
            return carry
        lax.fori_loop(0, nz, wout, 0)

        if with_cnt:
            pltpu.sync_copy(z_hbm, rows_v.at[0])
            pltpu.sync_copy(idn_hbm, idn_v)

            @pl.when(s == 0)
            def _():
                pltpu.sync_copy(z_hbm, acc_sh.at[pl.ds(0, CH)])
            hist = rows_v.at[0]
            ones16 = jnp.full((16,), 1.0, jnp.float32)

            def hloop(j, carry):
                for k in range(CH // 16):
                    idx16 = dst_v[j, pl.ds(k * 16, 16)]
                    plsc.addupdate_scatter(
                        hist,
                        [lax.shift_right_logical(idx16, 7),
                         lax.bitwise_and(idx16, 127)],
                        ones16)
                return carry
            lax.fori_loop(0, NCHUNK, hloop, 0)
            plsc.subcore_barrier()
            pltpu.sync_copy(hist, acc_sh.at[idn_v.at[0]], add=True)
            plsc.subcore_barrier()

            @pl.when(s == 0)
            def _():
                pltpu.sync_copy(acc_sh.at[pl.ds(0, CH)], cnt_hbm.at[c])

    return agg


_AGG_CNT = _make_sc_agg(True)
_AGG = _make_sc_agg(False)


def _linear0(x, wt, b):
    def body(x_ref, wt_ref, b_ref, o_ref):
        o_ref[...] = jnp.dot(x_ref[...], wt_ref[...],
                             preferred_element_type=jnp.float32) + b_ref[...]

    return pl.pallas_call(
        body,
        grid=(N // BR,),
        in_specs=[
            pl.BlockSpec((BR, D), lambda i: (i, 0)),
            pl.BlockSpec((D, D), lambda i: (0, 0)),
            pl.BlockSpec((1, D), lambda i: (0, 0)),
        ],
        out_specs=pl.BlockSpec((BR, D), lambda i: (i, 0)),
        out_shape=jax.ShapeDtypeStruct((N, D), jnp.float32),
    )(x, wt, b)


def _fused(p, ca, cb, g, gam, bet, wt=None, b=None):
    matmul = wt is not None

    def body(p_ref, ca_ref, cb_ref, g_ref, gam_ref, bet_ref, *rest):
        pv = p_ref[...]
        ssum = pv[0] + pv[1] + g_ref[...]
        den = ca_ref[...] + cb_ref[...] + 1.0
        a = jnp.maximum(ssum / den, 0.0)
        mu = jnp.mean(a, axis=1, keepdims=True)
        var = jnp.mean(jnp.square(a - mu), axis=1, keepdims=True)
        hn = (a - mu) / jnp.sqrt(var + 1e-5) * gam_ref[...] + bet_ref[...]
        if matmul:
            wt_ref, b_ref, o_ref = rest
            o_ref[...] = jnp.dot(hn, wt_ref[...],
                                 preferred_element_type=jnp.float32) + b_ref[...]
        else:
            (o_ref,) = rest
            o_ref[...] = hn

    in_specs = [
        pl.BlockSpec((2, BR, D), lambda i: (0, i, 0)),
        pl.BlockSpec((BR, 1), lambda i: (i, 0)),
        pl.BlockSpec((BR, 1), lambda i: (i, 0)),
        pl.BlockSpec((BR, D), lambda i: (i, 0)),
        pl.BlockSpec((1, D), lambda i: (0, 0)),
        pl.BlockSpec((1, D), lambda i: (0, 0)),
    ]
    args = [p, ca, cb, g, gam, bet]
    if matmul:
        in_specs += [pl.BlockSpec((D, D), lambda i: (0, 0)),
                     pl.BlockSpec((1, D), lambda i: (0, 0))]
        args += [wt, b]
    return pl.pallas_call(
        body,
        grid=(N // BR,),
        in_specs=in_specs,
        out_specs=pl.BlockSpec((BR, D), lambda i: (i, 0)),
        out_shape=jax.ShapeDtypeStruct((N, D), jnp.float32),
    )(*args)


def kernel(x, edge_index, W0, b0, W1, b1, W2, b2, g0, bt0, g1, bt1, g2, bt2):
    src3 = edge_index[0].reshape(NW, NCHUNK, 1, CH)
    dst3 = edge_index[1].reshape(NW, NCHUNK, CH)
    z = jnp.zeros((CH, D), jnp.float32)
    idn = jnp.arange(CH, dtype=jnp.int32).reshape(1, CH)
    b0r, b1r, b2r = b0.reshape(1, D), b1.reshape(1, D), b2.reshape(1, D)
    g0r, g1r, g2r = g0.reshape(1, D), g1.reshape(1, D), g2.reshape(1, D)
    bt0r, bt1r, bt2r = bt0.reshape(1, D), bt1.reshape(1, D), bt2.reshape(1, D)

    ga = _linear0(x, W0.T, b0r)
    p0, cnt = _AGG_CNT(ga, src3, dst3, z, idn)
    ca = cnt[0].reshape(CH * D, 1)[:N]
    cb = cnt[1].reshape(CH * D, 1)[:N]
    gb = _fused(p0, ca, cb, ga, g0r, bt0r, W1.T, b1r)
    p1 = _AGG(gb, src3, dst3, z, idn)
    gc = _fused(p1, ca, cb, gb, g1r, bt1r, W2.T, b2r)
    p2 = _AGG(gc, src3, dst3, z, idn)
    return _fused(p2, ca, cb, gc, g2r, bt2r)
